# baseline (device time: 39079 ns/iter reference)
import jax
import jax.numpy as jnp
from jax import lax
from jax.experimental import pallas as pl
from jax.experimental.pallas import tpu as pltpu

N = 16
P_W = 4
P_Z = 4
M = 1024
NCOL = 1024
ROWS = M // N
NH = 4
HR = ROWS // NH

F32 = jnp.float32
BF16 = jnp.bfloat16


def kernel(x):
    def body(x_ref, out_ref, stage, p1, csumf, csend, p2, colfin, pg,
             s1, r1, s2, r2, s3, r3, s4, r4):
        p = lax.axis_index("i")
        w = p % P_W
        z = p // P_W

        barrier = pltpu.get_barrier_semaphore()
        for j in range(1, P_W):
            pl.semaphore_signal(
                barrier, inc=1,
                device_id=(P_W * z + (w + j) % P_W,),
                device_id_type=pl.DeviceIdType.MESH,
            )
        for j in range(1, P_Z):
            pl.semaphore_signal(
                barrier, inc=1,
                device_id=(P_W * ((z + j) % P_Z) + w,),
                device_id_type=pl.DeviceIdType.MESH,
            )
        pl.semaphore_wait(barrier, 6)

        stage[...] = x_ref[0].astype(BF16).reshape(N, ROWS, NCOL)

        def rows(h):
            return pl.ds(h * HR, HR)

        def p1_sends(h):
            for j in range(1, P_W):
                wprime = (w + j) % P_W
                target = P_W * z + wprime
                for zp in range(P_Z):
                    c = P_W * zp + wprime
                    pltpu.make_async_remote_copy(
                        src_ref=stage.at[c, rows(h)],
                        dst_ref=p1.at[P_W - j, zp, rows(h)],
                        send_sem=s1.at[h, j, zp],
                        recv_sem=r1.at[h, P_W - j, zp],
                        device_id=(target,),
                        device_id_type=pl.DeviceIdType.MESH,
                    ).start()

        def p1_wait_reduce_p2send(h):
            for zp in range(P_Z):
                for o in range(1, P_W):
                    pltpu.make_async_remote_copy(
                        src_ref=stage.at[0, rows(h)],
                        dst_ref=p1.at[o, zp, rows(h)],
                        send_sem=s1.at[h, o, zp],
                        recv_sem=r1.at[h, o, zp],
                        device_id=(p,),
                        device_id_type=pl.DeviceIdType.MESH,
                    ).wait_recv()
                v = x_ref[0, pl.ds((P_W * zp + w) * ROWS + h * HR, HR), :]
                for o in range(1, P_W):
                    v = v + p1[o, zp, rows(h)].astype(F32)
                csumf[zp, rows(h)] = v
                csend[zp, rows(h)] = v.astype(BF16)

                @pl.when(zp != z)
                def _():
                    pltpu.make_async_remote_copy(
                        src_ref=csend.at[zp, rows(h)],
                        dst_ref=p2.at[(z - zp) % P_Z, rows(h)],
                        send_sem=s2.at[h, zp],
                        recv_sem=r2.at[h, (z - zp) % P_Z],
                        device_id=(P_W * zp + w,),
                        device_id_type=pl.DeviceIdType.MESH,
                    ).start()

        def p2_wait_final_p3send(h):
            fin = csumf[z, rows(h)]
            for o in range(1, P_Z):
                pltpu.make_async_remote_copy(
                    src_ref=csend.at[0, rows(h)],
                    dst_ref=p2.at[o, rows(h)],
                    send_sem=s2.at[h, 0],
                    recv_sem=r2.at[h, o],
                    device_id=(p,),
                    device_id_type=pl.DeviceIdType.MESH,
                ).wait_recv()
                fin = fin + p2[o, rows(h)].astype(F32)
            colfin[pl.ds(z, 1), rows(h)] = (
                fin.astype(BF16).reshape(1, HR, NCOL)
            )
            for j in range(1, P_Z):
                zpp = (z + j) % P_Z
                pltpu.make_async_remote_copy(
                    src_ref=colfin.at[z, rows(h)],
                    dst_ref=colfin.at[z, rows(h)],
                    send_sem=s3.at[h, j],
                    recv_sem=r3.at[h, z],
                    device_id=(P_W * zpp + w,),
                    device_id_type=pl.DeviceIdType.MESH,
                ).start()
            p3b_send_slice(h, z)

        def p3b_send_slice(h, zslot):
            for j in range(1, P_W):
                pltpu.make_async_remote_copy(
                    src_ref=colfin.at[zslot, rows(h)],
                    dst_ref=pg.at[P_W - j, zslot, rows(h)],
                    send_sem=s4.at[h, j, zslot],
                    recv_sem=r4.at[h, P_W - j, zslot],
                    device_id=(P_W * z + (w + j) % P_W,),
                    device_id_type=pl.DeviceIdType.MESH,
                ).start()

        def p3a_wait_forward(h):
            for o in range(1, P_Z):
                zsrc = (z + o) % P_Z
                pltpu.make_async_remote_copy(
                    src_ref=colfin.at[0, rows(h)],
                    dst_ref=colfin.at[zsrc, rows(h)],
                    send_sem=s3.at[h, 0],
                    recv_sem=r3.at[h, zsrc],
                    device_id=(p,),
                    device_id_type=pl.DeviceIdType.MESH,
                ).wait_recv()
                p3b_send_slice(h, zsrc)
            for zp in range(P_Z):
                out_ref[pl.ds((P_W * zp + w) * ROWS + h * HR, HR), :] = (
                    colfin[zp, rows(h)].astype(F32)
                )

        def p3b_wait_out(h):
            for o in range(1, P_W):
                for zp in range(P_Z):
                    pltpu.make_async_remote_copy(
                        src_ref=colfin.at[0, rows(h)],
                        dst_ref=pg.at[o, zp, rows(h)],
                        send_sem=s4.at[h, o, zp],
                        recv_sem=r4.at[h, o, zp],
                        device_id=(p,),
                        device_id_type=pl.DeviceIdType.MESH,
                    ).wait_recv()
                    out_ref[
                        pl.ds((P_W * zp + (w + o) % P_W) * ROWS + h * HR, HR),
                        :,
                    ] = pg[o, zp, rows(h)].astype(F32)

        for h in range(NH):
            p1_sends(h)
        for h in range(NH):
            p1_wait_reduce_p2send(h)
        for h in range(NH):
            p2_wait_final_p3send(h)
        for h in range(NH):
            p3a_wait_forward(h)
        for h in range(NH):
            p3b_wait_out(h)

        for h in range(NH):
            for j in range(1, P_W):
                for zp in range(P_Z):
                    pltpu.make_async_remote_copy(
                        src_ref=stage.at[0, rows(h)], dst_ref=p1.at[0, 0, rows(h)],
                        send_sem=s1.at[h, j, zp], recv_sem=r1.at[h, 0, 0],
                        device_id=(p,), device_id_type=pl.DeviceIdType.MESH,
                    ).wait_send()
                    pltpu.make_async_remote_copy(
                        src_ref=colfin.at[0, rows(h)], dst_ref=pg.at[0, 0, rows(h)],
                        send_sem=s4.at[h, j, zp], recv_sem=r4.at[h, 0, 0],
                        device_id=(p,), device_id_type=pl.DeviceIdType.MESH,
                    ).wait_send()
            for zp in range(P_Z):
                @pl.when(zp != z)
                def _():
                    pltpu.make_async_remote_copy(
                        src_ref=csend.at[0, rows(h)], dst_ref=p2.at[0, rows(h)],
                        send_sem=s2.at[h, zp], recv_sem=r2.at[h, 0],
                        device_id=(p,), device_id_type=pl.DeviceIdType.MESH,
                    ).wait_send()
            for j in range(1, P_Z):
                pltpu.make_async_remote_copy(
                    src_ref=colfin.at[0, rows(h)], dst_ref=colfin.at[0, rows(h)],
                    send_sem=s3.at[h, j], recv_sem=r3.at[h, 0],
                    device_id=(p,), device_id_type=pl.DeviceIdType.MESH,
                ).wait_send()

    return pl.pallas_call(
        body,
        out_shape=jax.ShapeDtypeStruct((M, NCOL), F32),
        in_specs=[pl.BlockSpec(memory_space=pltpu.VMEM)],
        out_specs=pl.BlockSpec(memory_space=pltpu.VMEM),
        scratch_shapes=[
            pltpu.VMEM((N, ROWS, NCOL), BF16),
            pltpu.VMEM((P_W, P_Z, ROWS, NCOL), BF16),
            pltpu.VMEM((P_Z, ROWS, NCOL), F32),
            pltpu.VMEM((P_Z, ROWS, NCOL), BF16),
            pltpu.VMEM((P_Z, ROWS, NCOL), BF16),
            pltpu.VMEM((P_Z, ROWS, NCOL), BF16),
            pltpu.VMEM((P_W, P_Z, ROWS, NCOL), BF16),
            pltpu.SemaphoreType.DMA((NH, P_W, P_Z)),
            pltpu.SemaphoreType.DMA((NH, P_W, P_Z)),
            pltpu.SemaphoreType.DMA((NH, P_Z)),
            pltpu.SemaphoreType.DMA((NH, P_Z)),
            pltpu.SemaphoreType.DMA((NH, P_Z)),
            pltpu.SemaphoreType.DMA((NH, P_Z)),
            pltpu.SemaphoreType.DMA((NH, P_W, P_Z)),
            pltpu.SemaphoreType.DMA((NH, P_W, P_Z)),
        ],
        compiler_params=pltpu.CompilerParams(collective_id=0),
    )(x)


# device time: 36377 ns/iter; 1.0743x vs baseline; 1.0743x over previous
import jax
import jax.numpy as jnp
from jax import lax
from jax.experimental import pallas as pl
from jax.experimental.pallas import tpu as pltpu

N = 16
P_W = 4
P_Z = 4
M = 1024
NCOL = 1024
ROWS = M // N
NH = 2
HR = ROWS // NH

F32 = jnp.float32
BF16 = jnp.bfloat16


def kernel(x):
    def body(x_ref, out_ref, stage, p1, csumf, csend, p2, colfin, pg,
             s1, r1, s2, r2, s3, r3, s4, r4):
        p = lax.axis_index("i")
        w = p % P_W
        z = p // P_W

        barrier = pltpu.get_barrier_semaphore()
        for j in range(1, P_W):
            pl.semaphore_signal(
                barrier, inc=1,
                device_id=(P_W * z + (w + j) % P_W,),
                device_id_type=pl.DeviceIdType.MESH,
            )
        for j in range(1, P_Z):
            pl.semaphore_signal(
                barrier, inc=1,
                device_id=(P_W * ((z + j) % P_Z) + w,),
                device_id_type=pl.DeviceIdType.MESH,
            )
        stage[...] = x_ref[0].astype(BF16).reshape(N, ROWS, NCOL)
        pl.semaphore_wait(barrier, 6)

        def rows(h):
            return pl.ds(h * HR, HR)

        def p1_sends(h):
            for j in range(1, P_W):
                wprime = (w + j) % P_W
                target = P_W * z + wprime
                for zp in range(P_Z):
                    c = P_W * zp + wprime
                    pltpu.make_async_remote_copy(
                        src_ref=stage.at[c, rows(h)],
                        dst_ref=p1.at[P_W - j, zp, rows(h)],
                        send_sem=s1.at[h, j, zp],
                        recv_sem=r1.at[h, P_W - j, zp],
                        device_id=(target,),
                        device_id_type=pl.DeviceIdType.MESH,
                    ).start()

        def p1_wait_reduce_p2send(h):
            for zp in range(P_Z):
                for o in range(1, P_W):
                    pltpu.make_async_remote_copy(
                        src_ref=stage.at[0, rows(h)],
                        dst_ref=p1.at[o, zp, rows(h)],
                        send_sem=s1.at[h, o, zp],
                        recv_sem=r1.at[h, o, zp],
                        device_id=(p,),
                        device_id_type=pl.DeviceIdType.MESH,
                    ).wait_recv()
                v = x_ref[0, pl.ds((P_W * zp + w) * ROWS + h * HR, HR), :]
                for o in range(1, P_W):
                    v = v + p1[o, zp, rows(h)].astype(F32)
                csumf[zp, rows(h)] = v
                csend[zp, rows(h)] = v.astype(BF16)

                @pl.when(zp != z)
                def _():
                    pltpu.make_async_remote_copy(
                        src_ref=csend.at[zp, rows(h)],
                        dst_ref=p2.at[(z - zp) % P_Z, rows(h)],
                        send_sem=s2.at[h, zp],
                        recv_sem=r2.at[h, (z - zp) % P_Z],
                        device_id=(P_W * zp + w,),
                        device_id_type=pl.DeviceIdType.MESH,
                    ).start()

        def p2_wait_final_p3send(h):
            fin = csumf[z, rows(h)]
            for o in range(1, P_Z):
                pltpu.make_async_remote_copy(
                    src_ref=csend.at[0, rows(h)],
                    dst_ref=p2.at[o, rows(h)],
                    send_sem=s2.at[h, 0],
                    recv_sem=r2.at[h, o],
                    device_id=(p,),
                    device_id_type=pl.DeviceIdType.MESH,
                ).wait_recv()
                fin = fin + p2[o, rows(h)].astype(F32)
            colfin[pl.ds(z, 1), rows(h)] = (
                fin.astype(BF16).reshape(1, HR, NCOL)
            )
            for j in range(1, P_Z):
                zpp = (z + j) % P_Z
                pltpu.make_async_remote_copy(
                    src_ref=colfin.at[z, rows(h)],
                    dst_ref=colfin.at[z, rows(h)],
                    send_sem=s3.at[h, j],
                    recv_sem=r3.at[h, z],
                    device_id=(P_W * zpp + w,),
                    device_id_type=pl.DeviceIdType.MESH,
                ).start()
            p3b_send_slice(h, z)

        def p3b_send_slice(h, zslot):
            for j in range(1, P_W):
                pltpu.make_async_remote_copy(
                    src_ref=colfin.at[zslot, rows(h)],
                    dst_ref=pg.at[P_W - j, zslot, rows(h)],
                    send_sem=s4.at[h, j, zslot],
                    recv_sem=r4.at[h, P_W - j, zslot],
                    device_id=(P_W * z + (w + j) % P_W,),
                    device_id_type=pl.DeviceIdType.MESH,
                ).start()

        def p3a_wait_forward(h):
            for o in range(1, P_Z):
                zsrc = (z + o) % P_Z
                pltpu.make_async_remote_copy(
                    src_ref=colfin.at[0, rows(h)],
                    dst_ref=colfin.at[zsrc, rows(h)],
                    send_sem=s3.at[h, 0],
                    recv_sem=r3.at[h, zsrc],
                    device_id=(p,),
                    device_id_type=pl.DeviceIdType.MESH,
                ).wait_recv()
                p3b_send_slice(h, zsrc)

        def out_own(h):
            for zp in range(P_Z):
                out_ref[pl.ds((P_W * zp + w) * ROWS + h * HR, HR), :] = (
                    colfin[zp, rows(h)]
                )

        def p3b_wait_out(h):
            for o in range(1, P_W):
                for zp in range(P_Z):
                    pltpu.make_async_remote_copy(
                        src_ref=colfin.at[0, rows(h)],
                        dst_ref=pg.at[o, zp, rows(h)],
                        send_sem=s4.at[h, o, zp],
                        recv_sem=r4.at[h, o, zp],
                        device_id=(p,),
                        device_id_type=pl.DeviceIdType.MESH,
                    ).wait_recv()
                    out_ref[
                        pl.ds((P_W * zp + (w + o) % P_W) * ROWS + h * HR, HR),
                        :,
                    ] = pg[o, zp, rows(h)]

        for h in range(NH):
            p1_sends(h)
        for h in range(NH):
            p1_wait_reduce_p2send(h)
            if h > 0:
                p3a_wait_forward(h - 1)
            p2_wait_final_p3send(h)
        p3a_wait_forward(NH - 1)
        for h in range(NH):
            p3b_wait_out(h)
            out_own(h)

        for h in range(NH):
            for j in range(1, P_W):
                for zp in range(P_Z):
                    pltpu.make_async_remote_copy(
                        src_ref=stage.at[0, rows(h)], dst_ref=p1.at[0, 0, rows(h)],
                        send_sem=s1.at[h, j, zp], recv_sem=r1.at[h, 0, 0],
                        device_id=(p,), device_id_type=pl.DeviceIdType.MESH,
                    ).wait_send()
                    pltpu.make_async_remote_copy(
                        src_ref=colfin.at[0, rows(h)], dst_ref=pg.at[0, 0, rows(h)],
                        send_sem=s4.at[h, j, zp], recv_sem=r4.at[h, 0, 0],
                        device_id=(p,), device_id_type=pl.DeviceIdType.MESH,
                    ).wait_send()
            for zp in range(P_Z):
                @pl.when(zp != z)
                def _():
                    pltpu.make_async_remote_copy(
                        src_ref=csend.at[0, rows(h)], dst_ref=p2.at[0, rows(h)],
                        send_sem=s2.at[h, zp], recv_sem=r2.at[h, 0],
                        device_id=(p,), device_id_type=pl.DeviceIdType.MESH,
                    ).wait_send()
            for j in range(1, P_Z):
                pltpu.make_async_remote_copy(
                    src_ref=colfin.at[0, rows(h)], dst_ref=colfin.at[0, rows(h)],
                    send_sem=s3.at[h, j], recv_sem=r3.at[h, 0],
                    device_id=(p,), device_id_type=pl.DeviceIdType.MESH,
                ).wait_send()

    return pl.pallas_call(
        body,
        out_shape=jax.ShapeDtypeStruct((M, NCOL), BF16),
        in_specs=[pl.BlockSpec(memory_space=pltpu.VMEM)],
        out_specs=pl.BlockSpec(memory_space=pltpu.VMEM),
        scratch_shapes=[
            pltpu.VMEM((N, ROWS, NCOL), BF16),
            pltpu.VMEM((P_W, P_Z, ROWS, NCOL), BF16),
            pltpu.VMEM((P_Z, ROWS, NCOL), F32),
            pltpu.VMEM((P_Z, ROWS, NCOL), BF16),
            pltpu.VMEM((P_Z, ROWS, NCOL), BF16),
            pltpu.VMEM((P_Z, ROWS, NCOL), BF16),
            pltpu.VMEM((P_W, P_Z, ROWS, NCOL), BF16),
            pltpu.SemaphoreType.DMA((NH, P_W, P_Z)),
            pltpu.SemaphoreType.DMA((NH, P_W, P_Z)),
            pltpu.SemaphoreType.DMA((NH, P_Z)),
            pltpu.SemaphoreType.DMA((NH, P_Z)),
            pltpu.SemaphoreType.DMA((NH, P_Z)),
            pltpu.SemaphoreType.DMA((NH, P_Z)),
            pltpu.SemaphoreType.DMA((NH, P_W, P_Z)),
            pltpu.SemaphoreType.DMA((NH, P_W, P_Z)),
        ],
        compiler_params=pltpu.CompilerParams(collective_id=0),
    )(x)
